# Initial kernel scaffold; baseline (speedup 1.0000x reference)
import jax
import jax.numpy as jnp
from jax.experimental import pallas as pl


def kernel(indices, table, lora_embedding_A, lora_embedding_B):
    raise NotImplementedError("write your pallas kernel here")


# speedup vs baseline: 9.5967x; 9.5967x over previous
import functools

import jax
import jax.numpy as jnp
from jax import lax
from jax.experimental import pallas as pl
from jax.experimental.pallas import tpu as pltpu
from jax.experimental.pallas import tpu_sc as plsc

V, D, R = 1_000_000, 64, 16
SCALING = 2.0

BV = 8000
VEC = 128
CHUNK_VECS = 8
CHUNK = CHUNK_VECS * VEC


def _merge_body(tab_ref, at_ref, bt_ref, out_ref):
    lora = lax.dot_general(
        at_ref[...], bt_ref[...],
        dimension_numbers=(((1,), (0,)), ((), ())),
        preferred_element_type=jnp.float32,
    )
    out_ref[...] = tab_ref[...] + SCALING * lora


def _merged_table(table, lora_at, lora_bt):
    return pl.pallas_call(
        _merge_body,
        grid=(V // BV,),
        in_specs=[
            pl.BlockSpec((BV, D), lambda i: (i, 0)),
            pl.BlockSpec((BV, R), lambda i: (i, 0)),
            pl.BlockSpec((R, D), lambda i: (0, 0)),
        ],
        out_specs=pl.BlockSpec((BV, D), lambda i: (i, 0)),
        out_shape=jax.ShapeDtypeStruct((V, D), jnp.float32),
    )(table, lora_at, lora_bt)


def _gather_rows(merged, idx2d, n_tokens):
    info = plsc.get_sparse_core_info()
    nw = info.num_cores * info.num_subcores
    vecs_total = idx2d.shape[0]
    vecs_per_w = vecs_total // nw
    steps = vecs_per_w // CHUNK_VECS
    mesh = plsc.VectorSubcoreMesh(core_axis_name="c", subcore_axis_name="s")

    @functools.partial(
        pl.kernel,
        mesh=mesh,
        out_type=jax.ShapeDtypeStruct((n_tokens, D), jnp.float32),
        scratch_types=[
            pltpu.VMEM((CHUNK_VECS, VEC), jnp.int32),
            pltpu.VMEM((CHUNK, D), jnp.float32),
            pltpu.SemaphoreType.DMA,
        ],
        compiler_params=pltpu.CompilerParams(use_tc_tiling_on_sc=False),
    )
    def k(idx_hbm, merged_hbm, out_hbm, idx_v, rows_v, sem):
        wid = lax.axis_index("s") * info.num_cores + lax.axis_index("c")
        vec_base = wid * vecs_per_w

        def step(g, carry):
            v0 = vec_base + g * CHUNK_VECS
            pltpu.sync_copy(idx_hbm.at[pl.ds(v0, CHUNK_VECS)], idx_v)
            copies = [
                pltpu.async_copy(
                    merged_hbm.at[idx_v.at[j]],
                    rows_v.at[pl.ds(j * VEC, VEC)],
                    sem,
                )
                for j in range(CHUNK_VECS)
            ]
            for c in copies:
                c.wait()
            pltpu.sync_copy(rows_v, out_hbm.at[pl.ds(v0 * VEC, CHUNK)])
            return carry

        lax.fori_loop(0, steps, step, None)

    return k(idx2d, merged)


def kernel(indices, table, lora_embedding_A, lora_embedding_B):
    b, l = indices.shape
    n = b * l
    merged = _merged_table(
        table, lora_embedding_A.T, lora_embedding_B.T)
    idx2d = indices.astype(jnp.int32).reshape(n // VEC, VEC)
    out = _gather_rows(merged, idx2d, n)
    return out.reshape(b, l, D)


# speedup vs baseline: 12.6376x; 1.3169x over previous
import functools

import jax
import jax.numpy as jnp
from jax import lax
from jax.experimental import pallas as pl
from jax.experimental.pallas import tpu as pltpu
from jax.experimental.pallas import tpu_sc as plsc

V, D, R = 1_000_000, 64, 16
SCALING = 2.0

MCHUNK = 8064
MSTEPS = 124
VTAIL = V - MSTEPS * MCHUNK
VEC = 128
CHUNK_VECS = 8
CHUNK = CHUNK_VECS * VEC


def _merge_step(bt_ref, eye_ref, tab_tail_ref, a_tail_ref,
                tt_hbm, a_hbm, out_hbm,
                tt_v, a_v, out_v, tail_v, in_sem, out_sem, tail_sem):
    i = pl.program_id(0)
    slot = lax.rem(i, 2)
    nxt = lax.rem(i + 1, 2)

    def start_in(step, s):
        off = step * MCHUNK
        pltpu.make_async_copy(
            tt_hbm.at[:, pl.ds(off, MCHUNK)], tt_v.at[s], in_sem.at[s]
        ).start()
        pltpu.make_async_copy(
            a_hbm.at[:, pl.ds(off, MCHUNK)], a_v.at[s], in_sem.at[s]
        ).start()

    def wait_in(s):
        pltpu.make_async_copy(
            tt_hbm.at[:, pl.ds(0, MCHUNK)], tt_v.at[s], in_sem.at[s]
        ).wait()
        pltpu.make_async_copy(
            a_hbm.at[:, pl.ds(0, MCHUNK)], a_v.at[s], in_sem.at[s]
        ).wait()

    def out_copy(step, s):
        return pltpu.make_async_copy(
            out_v.at[s], out_hbm.at[pl.ds(step * MCHUNK, MCHUNK)],
            out_sem.at[s],
        )

    @pl.when(i == 0)
    def _():
        start_in(0, 0)

    @pl.when(i + 1 < MSTEPS)
    def _():
        start_in(i + 1, nxt)

    wait_in(slot)

    tabt = lax.dot_general(
        tt_v[slot], eye_ref[...],
        dimension_numbers=(((0,), (0,)), ((), ())),
        preferred_element_type=jnp.float32,
    )
    lora = lax.dot_general(
        a_v[slot], bt_ref[...],
        dimension_numbers=(((0,), (0,)), ((), ())),
        preferred_element_type=jnp.float32,
    )

    @pl.when(i >= 2)
    def _():
        out_copy(i - 2, slot).wait()

    out_v[slot] = tabt + SCALING * lora
    out_copy(i, slot).start()

    @pl.when(i == MSTEPS - 1)
    def _():
        lora_tail = lax.dot_general(
            a_tail_ref[...], bt_ref[...],
            dimension_numbers=(((0,), (0,)), ((), ())),
            preferred_element_type=jnp.float32,
        )
        tail_v[...] = tab_tail_ref[...] + SCALING * lora_tail
        tail_cp = pltpu.make_async_copy(
            tail_v, out_hbm.at[pl.ds(MSTEPS * MCHUNK, VTAIL)], tail_sem)
        tail_cp.start()
        out_copy(i - 1, nxt).wait()
        out_copy(i, slot).wait()
        tail_cp.wait()


def _merge(table_t, lora_a, lora_bt, eye, tab_tail, a_tail):
    return pl.pallas_call(
        _merge_step,
        grid=(MSTEPS,),
        in_specs=[
            pl.BlockSpec((R, D), lambda i: (0, 0)),
            pl.BlockSpec((D, D), lambda i: (0, 0)),
            pl.BlockSpec((VTAIL, D), lambda i: (0, 0)),
            pl.BlockSpec((R, VTAIL), lambda i: (0, 0)),
            pl.BlockSpec(memory_space=pl.ANY),
            pl.BlockSpec(memory_space=pl.ANY),
        ],
        out_specs=pl.BlockSpec(memory_space=pl.ANY),
        out_shape=jax.ShapeDtypeStruct((V, D), jnp.float32),
        scratch_shapes=[
            pltpu.VMEM((2, D, MCHUNK), jnp.float32),
            pltpu.VMEM((2, R, MCHUNK), jnp.float32),
            pltpu.VMEM((2, MCHUNK, D), jnp.float32),
            pltpu.VMEM((VTAIL, D), jnp.float32),
            pltpu.SemaphoreType.DMA((2,)),
            pltpu.SemaphoreType.DMA((2,)),
            pltpu.SemaphoreType.DMA,
        ],
    )(lora_bt, eye, tab_tail, a_tail, table_t, lora_a)


def _gather_rows(merged, idx2d, n_tokens):
    info = plsc.get_sparse_core_info()
    nw = info.num_cores * info.num_subcores
    vecs_total = idx2d.shape[0]
    vecs_per_w = vecs_total // nw
    steps = vecs_per_w // CHUNK_VECS
    mesh = plsc.VectorSubcoreMesh(core_axis_name="c", subcore_axis_name="s")

    @functools.partial(
        pl.kernel,
        mesh=mesh,
        out_type=jax.ShapeDtypeStruct((n_tokens, D), jnp.float32),
        scratch_types=[
            pltpu.VMEM((CHUNK_VECS, VEC), jnp.int32),
            pltpu.VMEM((CHUNK, D), jnp.float32),
            pltpu.SemaphoreType.DMA,
        ],
        compiler_params=pltpu.CompilerParams(use_tc_tiling_on_sc=False),
    )
    def k(idx_hbm, merged_hbm, out_hbm, idx_v, rows_v, sem):
        wid = lax.axis_index("s") * info.num_cores + lax.axis_index("c")
        vec_base = wid * vecs_per_w

        def step(g, carry):
            v0 = vec_base + g * CHUNK_VECS
            pltpu.sync_copy(idx_hbm.at[pl.ds(v0, CHUNK_VECS)], idx_v)
            copies = [
                pltpu.async_copy(
                    merged_hbm.at[idx_v.at[j]],
                    rows_v.at[pl.ds(j * VEC, VEC)],
                    sem,
                )
                for j in range(CHUNK_VECS)
            ]
            for c in copies:
                c.wait()
            pltpu.sync_copy(rows_v, out_hbm.at[pl.ds(v0 * VEC, CHUNK)])
            return carry

        lax.fori_loop(0, steps, step, None)

    return k(idx2d, merged)


def kernel(indices, table, lora_embedding_A, lora_embedding_B):
    b, l = indices.shape
    n = b * l
    eye = jnp.eye(D, dtype=jnp.float32)
    tab_tail = lax.slice(table, (MSTEPS * MCHUNK, 0), (V, D))
    a_tail = lax.slice(lora_embedding_A, (0, MSTEPS * MCHUNK), (R, V))
    merged = _merge(
        table.T, lora_embedding_A, lora_embedding_B.T, eye, tab_tail, a_tail)
    idx2d = indices.astype(jnp.int32).reshape(n // VEC, VEC)
    out = _gather_rows(merged, idx2d, n)
    return out.reshape(b, l, D)


# speedup vs baseline: 13.5723x; 1.0740x over previous
import functools

import jax
import jax.numpy as jnp
from jax import lax
from jax.experimental import pallas as pl
from jax.experimental.pallas import tpu as pltpu
from jax.experimental.pallas import tpu_sc as plsc

V, D, R = 1_000_000, 64, 16
SCALING = 2.0

MCHUNK = 16128
MSTEPS = 62
VTAIL = V - MSTEPS * MCHUNK
VEC = 128
CHUNK_VECS = 8
CHUNK = CHUNK_VECS * VEC


def _merge_step(rhs_ref, tab_tail_ref, a_tail_ref,
                tt_hbm, a_hbm, out_hbm,
                cat_v, out_v, tail_v, in_sem, out_sem, tail_sem):
    i = pl.program_id(0)
    slot = lax.rem(i, 2)
    nxt = lax.rem(i + 1, 2)

    def start_in(step, s):
        off = step * MCHUNK
        pltpu.make_async_copy(
            tt_hbm.at[:, pl.ds(off, MCHUNK)],
            cat_v.at[s, pl.ds(0, D)], in_sem.at[s],
        ).start()
        pltpu.make_async_copy(
            a_hbm.at[:, pl.ds(off, MCHUNK)],
            cat_v.at[s, pl.ds(D, R)], in_sem.at[s],
        ).start()

    def wait_in(s):
        pltpu.make_async_copy(
            tt_hbm.at[:, pl.ds(0, MCHUNK)],
            cat_v.at[s, pl.ds(0, D)], in_sem.at[s],
        ).wait()
        pltpu.make_async_copy(
            a_hbm.at[:, pl.ds(0, MCHUNK)],
            cat_v.at[s, pl.ds(D, R)], in_sem.at[s],
        ).wait()

    def out_copy(step, s):
        return pltpu.make_async_copy(
            out_v.at[s],
            out_hbm.at[pl.ds(step * MCHUNK, MCHUNK)],
            out_sem.at[s],
        )

    @pl.when(i == 0)
    def _():
        start_in(0, 0)

    @pl.when(i + 1 < MSTEPS)
    def _():
        start_in(i + 1, nxt)

    wait_in(slot)

    res = lax.dot_general(
        cat_v[slot], rhs_ref[...],
        dimension_numbers=(((0,), (0,)), ((), ())),
        preferred_element_type=jnp.float32,
    )

    @pl.when(i >= 2)
    def _():
        out_copy(i - 2, slot).wait()

    out_v[slot] = res
    out_copy(i, slot).start()

    @pl.when(i == MSTEPS - 1)
    def _():
        lora_tail = lax.dot_general(
            a_tail_ref[...], rhs_ref[pl.ds(D, R), :],
            dimension_numbers=(((0,), (0,)), ((), ())),
            preferred_element_type=jnp.float32,
        )
        tail_v[...] = tab_tail_ref[...] + lora_tail
        tail_cp = pltpu.make_async_copy(
            tail_v,
            out_hbm.at[pl.ds(MSTEPS * MCHUNK, VTAIL)], tail_sem)
        tail_cp.start()
        out_copy(i - 1, nxt).wait()
        out_copy(i, slot).wait()
        tail_cp.wait()


def _merge(table_t, lora_a, rhs, tab_tail, a_tail):
    return pl.pallas_call(
        _merge_step,
        grid=(MSTEPS,),
        in_specs=[
            pl.BlockSpec((D + R, D), lambda i: (0, 0)),
            pl.BlockSpec((VTAIL, D), lambda i: (0, 0)),
            pl.BlockSpec((R, VTAIL), lambda i: (0, 0)),
            pl.BlockSpec(memory_space=pl.ANY),
            pl.BlockSpec(memory_space=pl.ANY),
        ],
        out_specs=pl.BlockSpec(memory_space=pl.ANY),
        out_shape=jax.ShapeDtypeStruct((V, D), jnp.float32),
        scratch_shapes=[
            pltpu.VMEM((2, D + R, MCHUNK), jnp.float32),
            pltpu.VMEM((2, MCHUNK, D), jnp.float32),
            pltpu.VMEM((VTAIL, D), jnp.float32),
            pltpu.SemaphoreType.DMA((2,)),
            pltpu.SemaphoreType.DMA((2,)),
            pltpu.SemaphoreType.DMA,
        ],
    )(rhs, tab_tail, a_tail, table_t, lora_a)


def _gather_rows(merged, idx2d, n_tokens):
    info = plsc.get_sparse_core_info()
    nw = info.num_cores * info.num_subcores
    vecs_total = idx2d.shape[0]
    vecs_per_w = vecs_total // nw
    steps = vecs_per_w // CHUNK_VECS
    mesh = plsc.VectorSubcoreMesh(core_axis_name="c", subcore_axis_name="s")

    @functools.partial(
        pl.kernel,
        mesh=mesh,
        out_type=jax.ShapeDtypeStruct((n_tokens, D), jnp.float32),
        scratch_types=[
            pltpu.VMEM((CHUNK_VECS, VEC), jnp.int32),
            pltpu.VMEM((CHUNK, D), jnp.float32),
            pltpu.SemaphoreType.DMA,
        ],
        compiler_params=pltpu.CompilerParams(use_tc_tiling_on_sc=False),
    )
    def k(idx_hbm, merged_hbm, out_hbm, idx_v, rows_v, sem):
        wid = lax.axis_index("s") * info.num_cores + lax.axis_index("c")
        vec_base = wid * vecs_per_w

        def step(g, carry):
            v0 = vec_base + g * CHUNK_VECS
            pltpu.sync_copy(idx_hbm.at[pl.ds(v0, CHUNK_VECS)], idx_v)
            copies = [
                pltpu.async_copy(
                    merged_hbm.at[idx_v.at[j]],
                    rows_v.at[pl.ds(j * VEC, VEC)],
                    sem,
                )
                for j in range(CHUNK_VECS)
            ]
            for c in copies:
                c.wait()
            pltpu.sync_copy(rows_v, out_hbm.at[pl.ds(v0 * VEC, CHUNK)])
            return carry

        lax.fori_loop(0, steps, step, None)

    return k(idx2d, merged)


def kernel(indices, table, lora_embedding_A, lora_embedding_B):
    b, l = indices.shape
    n = b * l
    rhs = jnp.concatenate(
        [jnp.eye(D, dtype=jnp.float32),
         SCALING * lora_embedding_B.T], axis=0)
    tab_tail = lax.slice(table, (MSTEPS * MCHUNK, 0), (V, D))
    a_tail = lax.slice(lora_embedding_A, (0, MSTEPS * MCHUNK), (R, V))
    merged = _merge(table.T, lora_embedding_A, rhs, tab_tail, a_tail)
    idx2d = indices.astype(jnp.int32).reshape(n // VEC, VEC)
    out = _gather_rows(merged, idx2d, n)
    return out.reshape(b, l, D)


# speedup vs baseline: 17.7262x; 1.3061x over previous
import functools

import jax
import jax.numpy as jnp
from jax import lax
from jax.experimental import pallas as pl
from jax.experimental.pallas import tpu as pltpu
from jax.experimental.pallas import tpu_sc as plsc

V, D, R = 1_000_000, 64, 16
SCALING = 2.0
DP = 2 * D

MCHUNK = 8064
MSTEPS = 124
VTAIL = V - MSTEPS * MCHUNK
VEC = 128
CHUNK_VECS = 4
CHUNK = CHUNK_VECS * VEC
BT = 2048


def _merge_step(rhs_ref, tab_tail_ref, a_tail_ref,
                tt_hbm, a_hbm, out_hbm,
                cat_v, out_v, tail_v, in_sem, out_sem, tail_sem):
    i = pl.program_id(0)
    slot = lax.rem(i, 2)
    nxt = lax.rem(i + 1, 2)

    def start_in(step, s):
        off = step * MCHUNK
        pltpu.make_async_copy(
            tt_hbm.at[:, pl.ds(off, MCHUNK)],
            cat_v.at[s, pl.ds(0, D)], in_sem.at[s],
        ).start()
        pltpu.make_async_copy(
            a_hbm.at[:, pl.ds(off, MCHUNK)],
            cat_v.at[s, pl.ds(D, R)], in_sem.at[s],
        ).start()

    def wait_in(s):
        pltpu.make_async_copy(
            tt_hbm.at[:, pl.ds(0, MCHUNK)],
            cat_v.at[s, pl.ds(0, D)], in_sem.at[s],
        ).wait()
        pltpu.make_async_copy(
            a_hbm.at[:, pl.ds(0, MCHUNK)],
            cat_v.at[s, pl.ds(D, R)], in_sem.at[s],
        ).wait()

    def out_copy(step, s):
        return pltpu.make_async_copy(
            out_v.at[s],
            out_hbm.at[pl.ds(step * MCHUNK, MCHUNK)],
            out_sem.at[s],
        )

    @pl.when(i == 0)
    def _():
        start_in(0, 0)

    @pl.when(i + 1 < MSTEPS)
    def _():
        start_in(i + 1, nxt)

    wait_in(slot)

    res = lax.dot_general(
        cat_v[slot], rhs_ref[...],
        dimension_numbers=(((0,), (0,)), ((), ())),
        preferred_element_type=jnp.float32,
    )

    @pl.when(i >= 2)
    def _():
        out_copy(i - 2, slot).wait()

    out_v[slot, :, pl.ds(0, D)] = res
    out_copy(i, slot).start()

    @pl.when(i == MSTEPS - 1)
    def _():
        lora_tail = lax.dot_general(
            a_tail_ref[...], rhs_ref[pl.ds(D, R), :],
            dimension_numbers=(((0,), (0,)), ((), ())),
            preferred_element_type=jnp.float32,
        )
        tail_v[:, pl.ds(0, D)] = tab_tail_ref[...] + lora_tail
        tail_cp = pltpu.make_async_copy(
            tail_v,
            out_hbm.at[pl.ds(MSTEPS * MCHUNK, VTAIL)], tail_sem)
        tail_cp.start()
        out_copy(i - 1, nxt).wait()
        out_copy(i, slot).wait()
        tail_cp.wait()


def _merge(table_t, lora_a, rhs, tab_tail, a_tail):
    return pl.pallas_call(
        _merge_step,
        grid=(MSTEPS,),
        in_specs=[
            pl.BlockSpec((D + R, D), lambda i: (0, 0)),
            pl.BlockSpec((VTAIL, D), lambda i: (0, 0)),
            pl.BlockSpec((R, VTAIL), lambda i: (0, 0)),
            pl.BlockSpec(memory_space=pl.ANY),
            pl.BlockSpec(memory_space=pl.ANY),
        ],
        out_specs=pl.BlockSpec(memory_space=pl.ANY),
        out_shape=jax.ShapeDtypeStruct((V, DP), jnp.float32),
        scratch_shapes=[
            pltpu.VMEM((2, D + R, MCHUNK), jnp.float32),
            pltpu.VMEM((2, MCHUNK, DP), jnp.float32),
            pltpu.VMEM((VTAIL, DP), jnp.float32),
            pltpu.SemaphoreType.DMA((2,)),
            pltpu.SemaphoreType.DMA((2,)),
            pltpu.SemaphoreType.DMA,
        ],
    )(rhs, tab_tail, a_tail, table_t, lora_a)


def _gather_rows(merged, idx2d, n_tokens):
    info = plsc.get_sparse_core_info()
    nw = info.num_cores * info.num_subcores
    vecs_total = idx2d.shape[0]
    vecs_per_w = vecs_total // nw
    steps = vecs_per_w // CHUNK_VECS
    mesh = plsc.VectorSubcoreMesh(core_axis_name="c", subcore_axis_name="s")

    @functools.partial(
        pl.kernel,
        mesh=mesh,
        out_type=jax.ShapeDtypeStruct((n_tokens, DP), jnp.float32),
        scratch_types=[
            pltpu.VMEM((CHUNK_VECS, VEC), jnp.int32),
            pltpu.VMEM((CHUNK, DP), jnp.float32),
            pltpu.SemaphoreType.DMA,
        ],
    )
    def k(idx_hbm, merged_hbm, out_hbm, idx_v, rows_v, sem):
        wid = lax.axis_index("s") * info.num_cores + lax.axis_index("c")
        vec_base = wid * vecs_per_w

        def step(g, carry):
            v0 = vec_base + g * CHUNK_VECS
            pltpu.sync_copy(idx_hbm.at[pl.ds(v0, CHUNK_VECS)], idx_v)
            copies = [
                pltpu.async_copy(
                    merged_hbm.at[idx_v.at[j]],
                    rows_v.at[pl.ds(j * VEC, VEC)],
                    sem,
                )
                for j in range(CHUNK_VECS)
            ]
            for c in copies:
                c.wait()
            pltpu.sync_copy(rows_v, out_hbm.at[pl.ds(v0 * VEC, CHUNK)])
            return carry

        lax.fori_loop(0, steps, step, None)

    return k(idx2d, merged)


def _relayout_step(in_ref, out_ref):
    x = in_ref[0, :, pl.ds(0, D)]
    out_ref[0] = x.T


def _relayout(sc3, n_b, n_l):
    return pl.pallas_call(
        _relayout_step,
        grid=(n_l, n_b // BT),
        in_specs=[
            pl.BlockSpec((1, BT, DP), lambda l, j: (l, j, 0)),
        ],
        out_specs=pl.BlockSpec((1, D, BT), lambda l, j: (l, 0, j)),
        out_shape=jax.ShapeDtypeStruct((n_l, D, n_b), jnp.float32),
    )(sc3)


def kernel(indices, table, lora_embedding_A, lora_embedding_B):
    b, l = indices.shape
    n = b * l
    rhs = jnp.concatenate(
        [jnp.eye(D, dtype=jnp.float32),
         SCALING * lora_embedding_B.T], axis=0)
    tab_tail = lax.slice(table, (MSTEPS * MCHUNK, 0), (V, D))
    a_tail = lax.slice(lora_embedding_A, (0, MSTEPS * MCHUNK), (R, V))
    merged = _merge(table.T, lora_embedding_A, rhs, tab_tail, a_tail)
    idx2d = indices.T.astype(jnp.int32).reshape(n // VEC, VEC)
    sc_out = _gather_rows(merged, idx2d, n)
    y = _relayout(sc_out.reshape(l, b, DP), b, l)
    return y.transpose(2, 0, 1)


# speedup vs baseline: 19.6655x; 1.1094x over previous
import functools

import jax
import jax.numpy as jnp
from jax import lax
from jax.experimental import pallas as pl
from jax.experimental.pallas import tpu as pltpu
from jax.experimental.pallas import tpu_sc as plsc

V, D, R = 1_000_000, 64, 16
SCALING = 2.0
DP = 2 * D

MCHUNK = 8064
MSTEPS = 124
VTAIL = V - MSTEPS * MCHUNK
VEC = 128
CHUNK_VECS = 4
CHUNK = CHUNK_VECS * VEC
BT = 4096


def _merge_step(rhs_ref, tab_tail_ref, a_tail_ref,
                tt_hbm, a_hbm, out_hbm,
                cat_v, out_v, tail_v, in_sem, out_sem, tail_sem):
    i = pl.program_id(0)
    slot = lax.rem(i, 2)
    nxt = lax.rem(i + 1, 2)

    def start_in(step, s):
        off = step * MCHUNK
        pltpu.make_async_copy(
            tt_hbm.at[:, pl.ds(off, MCHUNK)],
            cat_v.at[s, pl.ds(0, D)], in_sem.at[s],
        ).start()
        pltpu.make_async_copy(
            a_hbm.at[:, pl.ds(off, MCHUNK)],
            cat_v.at[s, pl.ds(D, R)], in_sem.at[s],
        ).start()

    def wait_in(s):
        pltpu.make_async_copy(
            tt_hbm.at[:, pl.ds(0, MCHUNK)],
            cat_v.at[s, pl.ds(0, D)], in_sem.at[s],
        ).wait()
        pltpu.make_async_copy(
            a_hbm.at[:, pl.ds(0, MCHUNK)],
            cat_v.at[s, pl.ds(D, R)], in_sem.at[s],
        ).wait()

    def out_copy(step, s):
        return pltpu.make_async_copy(
            out_v.at[s],
            out_hbm.at[pl.ds(step * MCHUNK, MCHUNK)],
            out_sem.at[s],
        )

    @pl.when(i == 0)
    def _():
        start_in(0, 0)

    @pl.when(i + 1 < MSTEPS)
    def _():
        start_in(i + 1, nxt)

    wait_in(slot)

    res = lax.dot_general(
        cat_v[slot], rhs_ref[...],
        dimension_numbers=(((0,), (0,)), ((), ())),
        preferred_element_type=jnp.float32,
    )

    @pl.when(i >= 2)
    def _():
        out_copy(i - 2, slot).wait()

    out_v[slot, :, pl.ds(0, D)] = res
    out_copy(i, slot).start()

    @pl.when(i == MSTEPS - 1)
    def _():
        lora_tail = lax.dot_general(
            a_tail_ref[...], rhs_ref[pl.ds(D, R), :],
            dimension_numbers=(((0,), (0,)), ((), ())),
            preferred_element_type=jnp.float32,
        )
        tail_v[:, pl.ds(0, D)] = tab_tail_ref[...] + lora_tail
        tail_cp = pltpu.make_async_copy(
            tail_v,
            out_hbm.at[pl.ds(MSTEPS * MCHUNK, VTAIL)], tail_sem)
        tail_cp.start()
        out_copy(i - 1, nxt).wait()
        out_copy(i, slot).wait()
        tail_cp.wait()


def _merge(table_t, lora_a, rhs, tab_tail, a_tail):
    return pl.pallas_call(
        _merge_step,
        grid=(MSTEPS,),
        in_specs=[
            pl.BlockSpec((D + R, D), lambda i: (0, 0)),
            pl.BlockSpec((VTAIL, D), lambda i: (0, 0)),
            pl.BlockSpec((R, VTAIL), lambda i: (0, 0)),
            pl.BlockSpec(memory_space=pl.ANY),
            pl.BlockSpec(memory_space=pl.ANY),
        ],
        out_specs=pl.BlockSpec(memory_space=pl.ANY),
        out_shape=jax.ShapeDtypeStruct((V, DP), jnp.float32),
        scratch_shapes=[
            pltpu.VMEM((2, D + R, MCHUNK), jnp.float32),
            pltpu.VMEM((2, MCHUNK, DP), jnp.float32),
            pltpu.VMEM((VTAIL, DP), jnp.float32),
            pltpu.SemaphoreType.DMA((2,)),
            pltpu.SemaphoreType.DMA((2,)),
            pltpu.SemaphoreType.DMA,
        ],
    )(rhs, tab_tail, a_tail, table_t, lora_a)


def _gather_rows(merged, idx2d, n_tokens):
    info = plsc.get_sparse_core_info()
    nw = info.num_cores * info.num_subcores
    vecs_total = idx2d.shape[0]
    vecs_per_w = vecs_total // nw
    steps = vecs_per_w // CHUNK_VECS
    mesh = plsc.VectorSubcoreMesh(core_axis_name="c", subcore_axis_name="s")

    @functools.partial(
        pl.kernel,
        mesh=mesh,
        out_type=jax.ShapeDtypeStruct((n_tokens, DP), jnp.float32),
        scratch_types=[
            pltpu.VMEM((CHUNK_VECS, VEC), jnp.int32),
            pltpu.VMEM((CHUNK, DP), jnp.float32),
            pltpu.SemaphoreType.DMA,
        ],
    )
    def k(idx_hbm, merged_hbm, out_hbm, idx_v, rows_v, sem):
        wid = lax.axis_index("s") * info.num_cores + lax.axis_index("c")
        vec_base = wid * vecs_per_w

        def step(g, carry):
            v0 = vec_base + g * CHUNK_VECS
            pltpu.sync_copy(idx_hbm.at[pl.ds(v0, CHUNK_VECS)], idx_v)
            copies = [
                pltpu.async_copy(
                    merged_hbm.at[idx_v.at[j]],
                    rows_v.at[pl.ds(j * VEC, VEC)],
                    sem,
                )
                for j in range(CHUNK_VECS)
            ]
            for c in copies:
                c.wait()
            pltpu.sync_copy(rows_v, out_hbm.at[pl.ds(v0 * VEC, CHUNK)])
            return carry

        lax.fori_loop(0, steps, step, None)

    return k(idx2d, merged)


def _relayout_step(in_ref, out_ref):
    x = in_ref[0, :, pl.ds(0, D)]
    out_ref[0] = x.T


def _relayout(sc3, n_b, n_l):
    return pl.pallas_call(
        _relayout_step,
        grid=(n_l, n_b // BT),
        in_specs=[
            pl.BlockSpec((1, BT, DP), lambda l, j: (l, j, 0)),
        ],
        out_specs=pl.BlockSpec((1, D, BT), lambda l, j: (l, 0, j)),
        out_shape=jax.ShapeDtypeStruct((n_l, D, n_b), jnp.float32),
    )(sc3)


def kernel(indices, table, lora_embedding_A, lora_embedding_B):
    b, l = indices.shape
    n = b * l
    rhs = jnp.concatenate(
        [jnp.eye(D, dtype=jnp.float32),
         SCALING * lora_embedding_B.T], axis=0)
    tab_tail = lax.slice(table, (MSTEPS * MCHUNK, 0), (V, D))
    a_tail = lax.slice(lora_embedding_A, (0, MSTEPS * MCHUNK), (R, V))
    merged = _merge(table.T, lora_embedding_A, rhs, tab_tail, a_tail)
    idx2d = indices.T.astype(jnp.int32).reshape(n // VEC, VEC)
    sc_out = _gather_rows(merged, idx2d, n)
    y = _relayout(sc_out.reshape(l, b, DP), b, l)
    return y.transpose(2, 0, 1)


# speedup vs baseline: 20.9910x; 1.0674x over previous
import functools

import jax
import jax.numpy as jnp
from jax import lax
from jax.experimental import pallas as pl
from jax.experimental.pallas import tpu as pltpu
from jax.experimental.pallas import tpu_sc as plsc

V, D, R = 1_000_000, 64, 16
SCALING = 2.0
DP = 2 * D

MCHUNK = 8064
MSTEPS = 124
VTAIL = V - MSTEPS * MCHUNK
VEC = 128
CHUNK_VECS = 4
CHUNK = CHUNK_VECS * VEC
BT = 8192


def _merge_step(rhs_ref, tab_tail_ref, a_tail_ref,
                tt_hbm, a_hbm, out_hbm,
                cat_v, out_v, tail_v, in_sem, out_sem, tail_sem):
    i = pl.program_id(0)
    slot = lax.rem(i, 2)
    nxt = lax.rem(i + 1, 2)

    def start_in(step, s):
        off = step * MCHUNK
        pltpu.make_async_copy(
            tt_hbm.at[:, pl.ds(off, MCHUNK)],
            cat_v.at[s, pl.ds(0, D)], in_sem.at[s],
        ).start()
        pltpu.make_async_copy(
            a_hbm.at[:, pl.ds(off, MCHUNK)],
            cat_v.at[s, pl.ds(D, R)], in_sem.at[s],
        ).start()

    def wait_in(s):
        pltpu.make_async_copy(
            tt_hbm.at[:, pl.ds(0, MCHUNK)],
            cat_v.at[s, pl.ds(0, D)], in_sem.at[s],
        ).wait()
        pltpu.make_async_copy(
            a_hbm.at[:, pl.ds(0, MCHUNK)],
            cat_v.at[s, pl.ds(D, R)], in_sem.at[s],
        ).wait()

    def out_copy(step, s):
        return pltpu.make_async_copy(
            out_v.at[s],
            out_hbm.at[pl.ds(step * MCHUNK, MCHUNK)],
            out_sem.at[s],
        )

    @pl.when(i == 0)
    def _():
        start_in(0, 0)

    @pl.when(i + 1 < MSTEPS)
    def _():
        start_in(i + 1, nxt)

    wait_in(slot)

    res = lax.dot_general(
        cat_v[slot], rhs_ref[...],
        dimension_numbers=(((0,), (0,)), ((), ())),
        preferred_element_type=jnp.float32,
    )

    @pl.when(i >= 2)
    def _():
        out_copy(i - 2, slot).wait()

    out_v[slot, :, pl.ds(0, D)] = res
    out_copy(i, slot).start()

    @pl.when(i == MSTEPS - 1)
    def _():
        lora_tail = lax.dot_general(
            a_tail_ref[...], rhs_ref[pl.ds(D, R), :],
            dimension_numbers=(((0,), (0,)), ((), ())),
            preferred_element_type=jnp.float32,
        )
        tail_v[:, pl.ds(0, D)] = tab_tail_ref[...] + lora_tail
        tail_cp = pltpu.make_async_copy(
            tail_v,
            out_hbm.at[pl.ds(MSTEPS * MCHUNK, VTAIL)], tail_sem)
        tail_cp.start()
        out_copy(i - 1, nxt).wait()
        out_copy(i, slot).wait()
        tail_cp.wait()


def _merge(table_t, lora_a, rhs, tab_tail, a_tail):
    return pl.pallas_call(
        _merge_step,
        grid=(MSTEPS,),
        in_specs=[
            pl.BlockSpec((D + R, D), lambda i: (0, 0)),
            pl.BlockSpec((VTAIL, D), lambda i: (0, 0)),
            pl.BlockSpec((R, VTAIL), lambda i: (0, 0)),
            pl.BlockSpec(memory_space=pl.ANY),
            pl.BlockSpec(memory_space=pl.ANY),
        ],
        out_specs=pl.BlockSpec(memory_space=pl.ANY),
        out_shape=jax.ShapeDtypeStruct((V, DP), jnp.float32),
        scratch_shapes=[
            pltpu.VMEM((2, D + R, MCHUNK), jnp.float32),
            pltpu.VMEM((2, MCHUNK, DP), jnp.float32),
            pltpu.VMEM((VTAIL, DP), jnp.float32),
            pltpu.SemaphoreType.DMA((2,)),
            pltpu.SemaphoreType.DMA((2,)),
            pltpu.SemaphoreType.DMA,
        ],
    )(rhs, tab_tail, a_tail, table_t, lora_a)


def _gather_rows(merged, idx2d, n_tokens):
    info = plsc.get_sparse_core_info()
    nw = info.num_cores * info.num_subcores
    vecs_total = idx2d.shape[0]
    vecs_per_w = vecs_total // nw
    steps = vecs_per_w // CHUNK_VECS
    mesh = plsc.VectorSubcoreMesh(core_axis_name="c", subcore_axis_name="s")

    @functools.partial(
        pl.kernel,
        mesh=mesh,
        out_type=jax.ShapeDtypeStruct((n_tokens, DP), jnp.float32),
        scratch_types=[
            pltpu.VMEM((CHUNK_VECS, VEC), jnp.int32),
            pltpu.VMEM((CHUNK, DP), jnp.float32),
            pltpu.SemaphoreType.DMA,
        ],
    )
    def k(idx_hbm, merged_hbm, out_hbm, idx_v, rows_v, sem):
        wid = lax.axis_index("s") * info.num_cores + lax.axis_index("c")
        vec_base = wid * vecs_per_w

        def step(g, carry):
            v0 = vec_base + g * CHUNK_VECS
            pltpu.sync_copy(idx_hbm.at[pl.ds(v0, CHUNK_VECS)], idx_v)
            copies = [
                pltpu.async_copy(
                    merged_hbm.at[idx_v.at[j]],
                    rows_v.at[pl.ds(j * VEC, VEC)],
                    sem,
                )
                for j in range(CHUNK_VECS)
            ]
            for c in copies:
                c.wait()
            pltpu.sync_copy(rows_v, out_hbm.at[pl.ds(v0 * VEC, CHUNK)])
            return carry

        lax.fori_loop(0, steps, step, None)

    return k(idx2d, merged)


def _relayout_step(in_ref, out_ref):
    x = in_ref[0, :, pl.ds(0, D)]
    out_ref[0] = x.T


def _relayout(sc3, n_b, n_l):
    return pl.pallas_call(
        _relayout_step,
        grid=(n_l, n_b // BT),
        in_specs=[
            pl.BlockSpec((1, BT, DP), lambda l, j: (l, j, 0)),
        ],
        out_specs=pl.BlockSpec((1, D, BT), lambda l, j: (l, 0, j)),
        out_shape=jax.ShapeDtypeStruct((n_l, D, n_b), jnp.float32),
    )(sc3)


def kernel(indices, table, lora_embedding_A, lora_embedding_B):
    b, l = indices.shape
    n = b * l
    rhs = jnp.concatenate(
        [jnp.eye(D, dtype=jnp.float32),
         SCALING * lora_embedding_B.T], axis=0)
    tab_tail = lax.slice(table, (MSTEPS * MCHUNK, 0), (V, D))
    a_tail = lax.slice(lora_embedding_A, (0, MSTEPS * MCHUNK), (R, V))
    merged = _merge(table.T, lora_embedding_A, rhs, tab_tail, a_tail)
    idx2d = indices.T.astype(jnp.int32).reshape(n // VEC, VEC)
    sc_out = _gather_rows(merged, idx2d, n)
    y = _relayout(sc_out.reshape(l, b, DP), b, l)
    return y.transpose(2, 0, 1)


# speedup vs baseline: 21.5720x; 1.0277x over previous
import functools

import jax
import jax.numpy as jnp
from jax import lax
from jax.experimental import pallas as pl
from jax.experimental.pallas import tpu as pltpu
from jax.experimental.pallas import tpu_sc as plsc

V, D, R = 1_000_000, 64, 16
SCALING = 2.0
DP = 2 * D

MCHUNK = 8064
MSTEPS = 124
VTAIL = V - MSTEPS * MCHUNK
VEC = 128
CHUNK_VECS = 4
CHUNK = CHUNK_VECS * VEC
BT = 8192


def _merge_step(rhs_ref, tab_tail_ref, a_tail_ref,
                tt_hbm, a_hbm, out_hbm,
                cat_v, out_v, tail_v, in_sem, out_sem, tail_sem):
    i = pl.program_id(0)
    slot = lax.rem(i, 2)
    nxt = lax.rem(i + 1, 2)

    def start_in(step, s):
        off = step * MCHUNK
        pltpu.make_async_copy(
            tt_hbm.at[:, pl.ds(off, MCHUNK)],
            cat_v.at[s, pl.ds(0, D)], in_sem.at[s],
        ).start()
        pltpu.make_async_copy(
            a_hbm.at[:, pl.ds(off, MCHUNK)],
            cat_v.at[s, pl.ds(D, R)], in_sem.at[s],
        ).start()

    def wait_in(s):
        pltpu.make_async_copy(
            tt_hbm.at[:, pl.ds(0, MCHUNK)],
            cat_v.at[s, pl.ds(0, D)], in_sem.at[s],
        ).wait()
        pltpu.make_async_copy(
            a_hbm.at[:, pl.ds(0, MCHUNK)],
            cat_v.at[s, pl.ds(D, R)], in_sem.at[s],
        ).wait()

    def out_copy(step, s):
        return pltpu.make_async_copy(
            out_v.at[s],
            out_hbm.at[pl.ds(step * MCHUNK, MCHUNK)],
            out_sem.at[s],
        )

    @pl.when(i == 0)
    def _():
        start_in(0, 0)

    @pl.when(i + 1 < MSTEPS)
    def _():
        start_in(i + 1, nxt)

    wait_in(slot)

    res = lax.dot_general(
        cat_v[slot], rhs_ref[...],
        dimension_numbers=(((0,), (0,)), ((), ())),
        preferred_element_type=jnp.float32,
    )

    @pl.when(i >= 2)
    def _():
        out_copy(i - 2, slot).wait()

    out_v[slot, :, pl.ds(0, D)] = res
    out_copy(i, slot).start()

    @pl.when(i == MSTEPS - 1)
    def _():
        lora_tail = lax.dot_general(
            a_tail_ref[...], rhs_ref[pl.ds(D, R), :],
            dimension_numbers=(((0,), (0,)), ((), ())),
            preferred_element_type=jnp.float32,
        )
        tail_v[:, pl.ds(0, D)] = tab_tail_ref[...] + lora_tail
        tail_cp = pltpu.make_async_copy(
            tail_v,
            out_hbm.at[pl.ds(MSTEPS * MCHUNK, VTAIL)], tail_sem)
        tail_cp.start()
        out_copy(i - 1, nxt).wait()
        out_copy(i, slot).wait()
        tail_cp.wait()


def _merge(table_t, lora_a, rhs, tab_tail, a_tail):
    return pl.pallas_call(
        _merge_step,
        grid=(MSTEPS,),
        in_specs=[
            pl.BlockSpec((D + R, D), lambda i: (0, 0)),
            pl.BlockSpec((VTAIL, D), lambda i: (0, 0)),
            pl.BlockSpec((R, VTAIL), lambda i: (0, 0)),
            pl.BlockSpec(memory_space=pl.ANY),
            pl.BlockSpec(memory_space=pl.ANY),
        ],
        out_specs=pl.BlockSpec(memory_space=pl.ANY),
        out_shape=jax.ShapeDtypeStruct((V, DP), jnp.float32),
        scratch_shapes=[
            pltpu.VMEM((2, D + R, MCHUNK), jnp.float32),
            pltpu.VMEM((2, MCHUNK, DP), jnp.float32),
            pltpu.VMEM((VTAIL, DP), jnp.float32),
            pltpu.SemaphoreType.DMA((2,)),
            pltpu.SemaphoreType.DMA((2,)),
            pltpu.SemaphoreType.DMA,
        ],
    )(rhs, tab_tail, a_tail, table_t, lora_a)


def _gather_rows(merged, idx2d, n_tokens):
    info = plsc.get_sparse_core_info()
    nw = info.num_cores * info.num_subcores
    vecs_total = idx2d.shape[0]
    vecs_per_w = vecs_total // nw
    steps = vecs_per_w // CHUNK_VECS
    mesh = plsc.VectorSubcoreMesh(core_axis_name="c", subcore_axis_name="s")

    @functools.partial(
        pl.kernel,
        mesh=mesh,
        out_type=jax.ShapeDtypeStruct((n_tokens, DP), jnp.float32),
        scratch_types=[
            pltpu.VMEM((CHUNK_VECS, VEC), jnp.int32),
            pltpu.VMEM((CHUNK, DP), jnp.float32),
            pltpu.SemaphoreType.DMA,
        ],
    )
    def k(idx_hbm, merged_hbm, out_hbm, idx_v, rows_v, sem):
        wid = lax.axis_index("s") * info.num_cores + lax.axis_index("c")
        vec_base = wid * vecs_per_w

        def step(g, carry):
            v0 = vec_base + g * CHUNK_VECS
            pltpu.sync_copy(idx_hbm.at[pl.ds(v0, CHUNK_VECS)], idx_v)
            copies = [
                pltpu.async_copy(
                    merged_hbm.at[idx_v.at[j]],
                    rows_v.at[pl.ds(j * VEC, VEC)],
                    sem,
                )
                for j in range(CHUNK_VECS)
            ]
            for c in copies:
                c.wait()
            pltpu.sync_copy(rows_v, out_hbm.at[pl.ds(v0 * VEC, CHUNK)])
            return carry

        lax.fori_loop(0, steps, step, None)

    return k(idx2d, merged)


def _relayout_step(in_ref, out_ref):
    x = in_ref[0, :, pl.ds(0, D)]
    out_ref[0] = x.T


def _relayout_half_first(sc3, n_b, n_l, l_cnt):
    return pl.pallas_call(
        _relayout_step,
        grid=(l_cnt, n_b // BT),
        in_specs=[
            pl.BlockSpec((1, BT, DP), lambda l, j: (l, j, 0)),
        ],
        out_specs=pl.BlockSpec((1, D, BT), lambda l, j: (l, 0, j)),
        out_shape=jax.ShapeDtypeStruct((n_l, D, n_b), jnp.float32),
    )(sc3)


def _relayout_half_second(sc3, y_prev, n_b, n_l, l_off):
    def step(in_ref, prev_ref, out_ref):
        _relayout_step(in_ref, out_ref)

    l_cnt = n_l - l_off
    return pl.pallas_call(
        step,
        grid=(l_cnt, n_b // BT),
        in_specs=[
            pl.BlockSpec((1, BT, DP), lambda l, j: (l, j, 0)),
            pl.BlockSpec(memory_space=pl.ANY),
        ],
        out_specs=pl.BlockSpec((1, D, BT), lambda l, j: (l + l_off, 0, j)),
        out_shape=jax.ShapeDtypeStruct((n_l, D, n_b), jnp.float32),
        input_output_aliases={1: 0},
    )(sc3, y_prev)


def kernel(indices, table, lora_embedding_A, lora_embedding_B):
    b, l = indices.shape
    n = b * l
    rhs = jnp.concatenate(
        [jnp.eye(D, dtype=jnp.float32),
         SCALING * lora_embedding_B.T], axis=0)
    tab_tail = lax.slice(table, (MSTEPS * MCHUNK, 0), (V, D))
    a_tail = lax.slice(lora_embedding_A, (0, MSTEPS * MCHUNK), (R, V))
    merged = _merge(table.T, lora_embedding_A, rhs, tab_tail, a_tail)
    idx2d = indices.T.astype(jnp.int32).reshape(n // VEC, VEC)
    lh = l // 2
    nh = lh * b
    vh = nh // VEC
    sc1 = _gather_rows(merged, lax.slice(idx2d, (0, 0), (vh, VEC)), nh)
    sc2 = _gather_rows(
        merged, lax.slice(idx2d, (vh, 0), (2 * vh, VEC)), nh)
    y1 = _relayout_half_first(sc1.reshape(lh, b, DP), b, l, lh)
    y = _relayout_half_second(sc2.reshape(lh, b, DP), y1, b, l, lh)
    return y.transpose(2, 0, 1)
